# baseline (speedup 1.0000x reference)
import jax
import jax.numpy as jnp
from jax.experimental import pallas as pl
from jax.experimental.pallas import tpu as pltpu

N_GROUPS = 9
N_PER_GROUP = 131072
C = 64
BLK = 16384
CPG = N_PER_GROUP // BLK
T = N_GROUPS * CPG
NBUF = 4


def _affine_kernel(x_hbm, w_vmem, b_vmem, o_hbm, x_vmem, y_vmem, in_sem, out_sem):
    t = pl.program_id(0)
    slot = jax.lax.rem(t, NBUF)
    g = jax.lax.div(t, CPG)

    def in_copy(chunk, s):
        cg = jax.lax.div(chunk, CPG)
        cn = jax.lax.rem(chunk, CPG)
        return pltpu.make_async_copy(
            x_hbm.at[cg, :, pl.ds(cn * BLK, BLK)],
            x_vmem.at[s],
            in_sem.at[s],
        )

    def out_copy(chunk, s):
        return pltpu.make_async_copy(
            y_vmem.at[s],
            o_hbm.at[:, pl.ds(chunk * BLK, BLK)],
            out_sem.at[s],
        )

    @pl.when(t == 0)
    def _prologue():
        for s in range(NBUF):
            in_copy(jnp.int32(s), jnp.int32(s)).start()

    in_copy(t, slot).wait()

    x = x_vmem[slot]
    w = w_vmem[g]
    b = b_vmem[g, 0]
    yt = jax.lax.dot_general(
        w, x, (((1,), (0,)), ((), ())), preferred_element_type=jnp.float32
    ) + b[:, None]

    @pl.when(t >= NBUF)
    def _wait_prev_out():
        out_copy(t - NBUF, slot).wait()

    y_vmem[slot] = yt
    out_copy(t, slot).start()

    @pl.when(t + NBUF < T)
    def _next_in():
        in_copy(t + NBUF, slot).start()

    @pl.when(t == T - 1)
    def _epilogue():
        for s in range(NBUF):
            c = T - NBUF + s
            out_copy(c, s).wait()


def kernel(inputs, weights, bias):
    x_t = jnp.transpose(inputs, (0, 2, 1))
    bias3 = bias.reshape(N_GROUPS, 1, C)
    out_t = pl.pallas_call(
        _affine_kernel,
        grid=(T,),
        in_specs=[
            pl.BlockSpec(memory_space=pl.ANY),
            pl.BlockSpec(memory_space=pltpu.VMEM),
            pl.BlockSpec(memory_space=pltpu.VMEM),
        ],
        out_specs=pl.BlockSpec(memory_space=pl.ANY),
        out_shape=jax.ShapeDtypeStruct((C, N_GROUPS * N_PER_GROUP), jnp.float32),
        scratch_shapes=[
            pltpu.VMEM((NBUF, C, BLK), jnp.float32),
            pltpu.VMEM((NBUF, C, BLK), jnp.float32),
            pltpu.SemaphoreType.DMA((NBUF,)),
            pltpu.SemaphoreType.DMA((NBUF,)),
        ],
    )(x_t, weights, bias3)
    return out_t.T
